# baseline (speedup 1.0000x reference)
import functools
import math

import jax
import jax.numpy as jnp
from jax.experimental import pallas as pl
from jax.experimental.pallas import tpu as pltpu

S = 4096
IN = 1024
H = 16
D = 64
ED = H * D
C = 100
K = 8
SB = 512
NSB = S // SB
HB = 2
H2 = H // HB
WKB = 512
NHALF = IN // WKB
T_B = NSB
T_C = NSB + K * NHALF
T_TOT = T_C + H2

bf16 = jnp.bfloat16
f32 = jnp.float32


def _ln(x, g, b, eps=1e-5):
    m = jnp.mean(x, axis=-1, keepdims=True)
    d = x - m
    v = jnp.mean(d * d, axis=-1, keepdims=True)
    return d * jax.lax.rsqrt(v + eps) * g + b


def _bdot(a, b):
    return jnp.dot(a.astype(bf16), b.astype(bf16), preferred_element_type=f32)


def _mega_kernel(x_ref, wih_ref, bih_ref, wq_ref, bq_ref, gq_ref, bbq_ref,
                 wv_ref, bv_ref, gv_ref, bbv_ref, wk_ref, bk_ref, gk_ref,
                 bbk_ref, wp_ref, bp_ref, go_ref, bbo_ref,
                 c_ref, idx_ref, attn_ref, o_ref,
                 xscr, cacc, qscr, vscr, idxscr, gscr, kacc, kscr, oacc):
    t = pl.program_id(0)

    @pl.when(t < T_B)
    def _phase_a():
        xb = x_ref[...]
        xscr[pl.ds(t * SB, SB), :] = xb.astype(bf16)
        cb = jnp.dot(xb, wih_ref[...], preferred_element_type=f32) + bih_ref[...]
        c_ref[...] = cb
        cacc[pl.ds(t * SB, SB), :] = cb
        qp = _bdot(xb, wq_ref[...]) + bq_ref[...]
        qln = _ln(qp, gq_ref[...], bbq_ref[...]).astype(bf16)
        qscr[:, pl.ds(t * SB, SB), :] = jnp.transpose(
            qln.reshape(SB, H2, HB * D), (1, 0, 2))
        vp = _bdot(xb, wv_ref[...]) + bv_ref[...]
        vln = _ln(vp, gv_ref[...], bbv_ref[...]).astype(bf16)
        vscr[:, pl.ds(t * SB, SB), :] = jnp.transpose(
            vln.reshape(SB, H2, HB * D), (1, 0, 2))

        @pl.when(t == T_B - 1)
        def _topk():
            iota = jax.lax.broadcasted_iota(jnp.int32, (S, C), 0)
            iota8 = jax.lax.broadcasted_iota(jnp.int32, (K, C), 0)

            def body(k, carry):
                cv, idxacc = carry
                m = jnp.max(cv, axis=0, keepdims=True)
                idx = jnp.min(jnp.where(cv >= m, iota, S), axis=0)
                idxacc = jnp.where(iota8 == k, idx[None, :], idxacc)
                cv = jnp.where(iota == idx[None, :], -jnp.inf, cv)
                return cv, idxacc

            _, idxacc = jax.lax.fori_loop(
                0, K, body, (cacc[...], jnp.zeros((K, C), jnp.int32)))
            idxscr[...] = idxacc
            idx_ref[...] = idxacc

    @pl.when(jnp.logical_and(t >= T_B, t < T_C))
    def _phase_b():
        jj = t - T_B
        j = jj // NHALF
        half = jj % NHALF

        @pl.when(half == 0)
        def _gather():
            idx_j = idxscr[pl.ds(j, 1), :]
            onehot = (jax.lax.broadcasted_iota(jnp.int32, (C, S), 1)
                      == jnp.reshape(idx_j, (C, 1))).astype(bf16)
            g = jnp.dot(onehot, xscr[...], preferred_element_type=f32)
            gscr[...] = jnp.transpose(
                g.astype(bf16).reshape(C, NHALF, WKB), (1, 0, 2))

        gh = gscr[pl.ds(half, 1), :, :].reshape(C, WKB)
        contrib = jnp.dot(gh, wk_ref[...].astype(bf16),
                          preferred_element_type=f32)

        @pl.when(jj == 0)
        def _():
            kacc[...] = contrib

        @pl.when(jj > 0)
        def _():
            kacc[...] += contrib

        @pl.when(jj == K * NHALF - 1)
        def _():
            pre = kacc[...] + bk_ref[...]
            kf = _ln(pre, gk_ref[...], bbk_ref[...]).astype(bf16)
            kscr[...] = jnp.transpose(kf.reshape(C, H, D), (1, 0, 2))

    @pl.when(t >= T_C)
    def _phase_c():
        h2 = t - T_C
        qpair = qscr[pl.ds(h2, 1), :, :].reshape(S, HB * D)
        vpair = vscr[pl.ds(h2, 1), :, :].reshape(S, HB * D)
        wpb = wp_ref[...]
        scale = 1.0 / math.sqrt(D)

        @pl.when(t == T_C)
        def _():
            oacc[...] = jnp.zeros((C, IN), f32)

        for hh in range(HB):
            qh = qpair[:, hh * D:(hh + 1) * D]
            vh = vpair[:, hh * D:(hh + 1) * D]
            kh = kscr[pl.ds(HB * h2 + hh, 1), :, :].reshape(C, D)
            logits = jax.lax.dot_general(qh, kh, (((1,), (1,)), ((), ())),
                                         preferred_element_type=f32) * scale
            e = jnp.exp(logits)
            a = e / jnp.sum(e, axis=1, keepdims=True)
            attn_ref[hh, :, :] = a
            contrib = jax.lax.dot_general(a.astype(bf16), vh, (((0,), (0,)), ((), ())),
                                          preferred_element_type=f32)
            oacc[...] += jnp.dot(contrib, wpb[hh * D:(hh + 1) * D, :],
                                 preferred_element_type=f32)

        @pl.when(t == T_TOT - 1)
        def _finish():
            pre = oacc[...] + bp_ref[...]
            o_ref[...] = _ln(pre, go_ref[...], bbo_ref[...])


def kernel(x, W_ih, b_ih, W_k, b_k, g_k, bb_k, W_q, b_q, g_q, bb_q,
           W_v, b_v, g_v, bb_v, W_p, b_p, g_o, bb_o):
    b_ih2 = b_ih.reshape(1, C)
    b_q2, g_q2, bb_q2 = b_q.reshape(1, ED), g_q.reshape(1, ED), bb_q.reshape(1, ED)
    b_v2, g_v2, bb_v2 = b_v.reshape(1, ED), g_v.reshape(1, ED), bb_v.reshape(1, ED)
    b_k2, g_k2, bb_k2 = b_k.reshape(1, ED), g_k.reshape(1, ED), bb_k.reshape(1, ED)
    b_p2, g_o2, bb_o2 = b_p.reshape(1, IN), g_o.reshape(1, IN), bb_o.reshape(1, IN)

    full = lambda shape: pl.BlockSpec(shape, lambda *_: tuple(0 for _ in shape))

    c, topk_idx, attn, o = pl.pallas_call(
        _mega_kernel,
        grid=(T_TOT,),
        in_specs=[
            pl.BlockSpec((SB, IN), lambda t: (jnp.minimum(t, NSB - 1), 0)),
            full((IN, C)), full((1, C)),
            full((IN, ED)), full((1, ED)), full((1, ED)), full((1, ED)),
            full((IN, ED)), full((1, ED)), full((1, ED)), full((1, ED)),
            pl.BlockSpec((WKB, ED), lambda t: (jnp.clip(t - T_B, 0, K * NHALF - 1), 0)),
            full((1, ED)), full((1, ED)), full((1, ED)),
            pl.BlockSpec((HB * D, IN), lambda t: (jnp.clip(t - T_C, 0, H2 - 1), 0)),
            full((1, IN)), full((1, IN)), full((1, IN)),
        ],
        out_specs=[
            pl.BlockSpec((SB, C), lambda t: (jnp.minimum(t, NSB - 1), 0)),
            full((K, C)),
            pl.BlockSpec((HB, S, C), lambda t: (jnp.clip(t - T_C, 0, H2 - 1), 0, 0)),
            full((C, IN)),
        ],
        out_shape=[
            jax.ShapeDtypeStruct((S, C), f32),
            jax.ShapeDtypeStruct((K, C), jnp.int32),
            jax.ShapeDtypeStruct((H, S, C), f32),
            jax.ShapeDtypeStruct((C, IN), f32),
        ],
        scratch_shapes=[
            pltpu.VMEM((S, IN), bf16),
            pltpu.VMEM((S, C), f32),
            pltpu.VMEM((H2, S, HB * D), bf16),
            pltpu.VMEM((H2, S, HB * D), bf16),
            pltpu.VMEM((K, C), jnp.int32),
            pltpu.VMEM((NHALF, C, WKB), bf16),
            pltpu.VMEM((C, ED), f32),
            pltpu.VMEM((H, C, D), bf16),
            pltpu.VMEM((C, IN), f32),
        ],
        compiler_params=pltpu.CompilerParams(
            vmem_limit_bytes=110 * 1024 * 1024,
        ),
    )(x, W_ih, b_ih2, W_q, b_q2, g_q2, bb_q2, W_v, b_v2, g_v2, bb_v2,
      W_k, b_k2, g_k2, bb_k2, W_p, b_p2, g_o2, bb_o2)

    return (o, c, attn, topk_idx)


# speedup vs baseline: 1.1593x; 1.1593x over previous
import functools
import math

import jax
import jax.numpy as jnp
from jax.experimental import pallas as pl
from jax.experimental.pallas import tpu as pltpu

S = 4096
IN = 1024
H = 16
D = 64
ED = H * D
C = 100
K = 8
SB = 1024
HB = 2
H2 = H // HB

bf16 = jnp.bfloat16
f32 = jnp.float32


def _ln(x, g, b, eps=1e-5):
    m = jnp.mean(x, axis=-1, keepdims=True)
    d = x - m
    v = jnp.mean(d * d, axis=-1, keepdims=True)
    return d * jax.lax.rsqrt(v + eps) * g + b


def _bdot(a, b):
    return jnp.dot(a.astype(bf16), b.astype(bf16), preferred_element_type=f32)


def _proj_kernel(x_ref, wih_ref, bih_ref, wq_ref, bq_ref, gq_ref, bbq_ref,
                 wv_ref, bv_ref, gv_ref, bbv_ref,
                 c_ref, q_ref, v_ref, idx_ref, cacc_ref):
    i = pl.program_id(0)
    xb = x_ref[...]
    cb = jnp.dot(xb, wih_ref[...], preferred_element_type=f32) + bih_ref[...]
    c_ref[...] = cb
    cacc_ref[pl.ds(i * SB, SB), :] = cb
    qp = _bdot(xb, wq_ref[...]) + bq_ref[...]
    q_ref[...] = _ln(qp, gq_ref[...], bbq_ref[...]).astype(bf16)
    vp = _bdot(xb, wv_ref[...]) + bv_ref[...]
    v_ref[...] = _ln(vp, gv_ref[...], bbv_ref[...]).astype(bf16)

    @pl.when(i == (S // SB) - 1)
    def _topk():
        cv = cacc_ref[...]
        iota = jax.lax.broadcasted_iota(jnp.int32, (S, C), 0)
        for k in range(K):
            m = jnp.max(cv, axis=0, keepdims=True)
            idx = jnp.min(jnp.where(cv >= m, iota, S), axis=0)
            idx_ref[k, :] = idx
            cv = jnp.where(iota == idx[None, :], -jnp.inf, cv)


def _kproj_kernel(idx_ref, x_ref, wk_ref, bk_ref, gk_ref, bbk_ref, k_ref, acc_ref):
    j = pl.program_id(0)
    idx_j = idx_ref[pl.ds(j, 1), :]
    onehot = (jax.lax.broadcasted_iota(jnp.int32, (C, S), 1)
              == jnp.reshape(idx_j, (C, 1))).astype(bf16)
    g = jnp.dot(onehot, x_ref[...].astype(bf16),
                preferred_element_type=f32)
    contrib = _bdot(g, wk_ref[...])

    @pl.when(j == 0)
    def _():
        acc_ref[...] = contrib

    @pl.when(j > 0)
    def _():
        acc_ref[...] += contrib

    @pl.when(j == K - 1)
    def _():
        pre = acc_ref[...] + bk_ref[...]
        kf = _ln(pre, gk_ref[...], bbk_ref[...]).astype(bf16)
        k_ref[...] = jnp.transpose(kf.reshape(C, H, D), (1, 0, 2))


def _attn_kernel(q_ref, k_ref, v_ref, attn_ref, val_ref):
    qb = q_ref[...]
    vb = v_ref[...]
    scale = 1.0 / math.sqrt(D)
    for hh in range(HB):
        qh = qb[:, hh * D:(hh + 1) * D]
        vh = vb[:, hh * D:(hh + 1) * D]
        kh = k_ref[hh]
        logits = jax.lax.dot_general(qh, kh, (((1,), (1,)), ((), ())),
                                     preferred_element_type=f32) * scale
        e = jnp.exp(logits)
        a = e / jnp.sum(e, axis=1, keepdims=True)
        attn_ref[hh, :, :] = a
        contrib = jax.lax.dot_general(a.astype(bf16), vh, (((0,), (0,)), ((), ())),
                                      preferred_element_type=f32)
        val_ref[hh, :, :] = contrib


def _out_kernel(val_ref, wp_ref, bp_ref, go_ref, bbo_ref, o_ref):
    vals = val_ref[...]
    acc = jnp.zeros((C, IN), f32)
    for h in range(H):
        acc += jnp.dot(vals[h], wp_ref[pl.ds(h * D, D), :],
                       preferred_element_type=f32)
    pre = acc + bp_ref[...]
    o_ref[...] = _ln(pre, go_ref[...], bbo_ref[...])


def kernel(x, W_ih, b_ih, W_k, b_k, g_k, bb_k, W_q, b_q, g_q, bb_q,
           W_v, b_v, g_v, bb_v, W_p, b_p, g_o, bb_o):
    b_ih2 = b_ih.reshape(1, C)
    b_q2, g_q2, bb_q2 = b_q.reshape(1, ED), g_q.reshape(1, ED), bb_q.reshape(1, ED)
    b_v2, g_v2, bb_v2 = b_v.reshape(1, ED), g_v.reshape(1, ED), bb_v.reshape(1, ED)
    b_k2, g_k2, bb_k2 = b_k.reshape(1, ED), g_k.reshape(1, ED), bb_k.reshape(1, ED)
    b_p2, g_o2, bb_o2 = b_p.reshape(1, IN), g_o.reshape(1, IN), bb_o.reshape(1, IN)

    full = lambda shape: pl.BlockSpec(shape, lambda *_: tuple(0 for _ in shape))

    nsb = S // SB
    c, q_, v_, topk_idx = pl.pallas_call(
        _proj_kernel,
        grid=(nsb,),
        in_specs=[
            pl.BlockSpec((SB, IN), lambda i: (i, 0)),
            full((IN, C)), full((1, C)),
            full((IN, ED)), full((1, ED)), full((1, ED)), full((1, ED)),
            full((IN, ED)), full((1, ED)), full((1, ED)), full((1, ED)),
        ],
        out_specs=[
            pl.BlockSpec((SB, C), lambda i: (i, 0)),
            pl.BlockSpec((SB, ED), lambda i: (i, 0)),
            pl.BlockSpec((SB, ED), lambda i: (i, 0)),
            full((K, C)),
        ],
        out_shape=[
            jax.ShapeDtypeStruct((S, C), f32),
            jax.ShapeDtypeStruct((S, ED), bf16),
            jax.ShapeDtypeStruct((S, ED), bf16),
            jax.ShapeDtypeStruct((K, C), jnp.int32),
        ],
        scratch_shapes=[pltpu.VMEM((S, C), f32)],
    )(x, W_ih, b_ih2, W_q, b_q2, g_q2, bb_q2, W_v, b_v2, g_v2, bb_v2)

    k_ = pl.pallas_call(
        _kproj_kernel,
        grid=(K,),
        in_specs=[
            full((K, C)),
            full((S, IN)),
            pl.BlockSpec((IN, ED), lambda j: (j, 0)),
            full((1, ED)), full((1, ED)), full((1, ED)),
        ],
        out_specs=pl.BlockSpec((H, C, D), lambda j: (0, 0, 0)),
        out_shape=jax.ShapeDtypeStruct((H, C, D), bf16),
        scratch_shapes=[pltpu.VMEM((C, ED), f32)],
    )(topk_idx, x, W_k, b_k2, g_k2, bb_k2)

    attn, values = pl.pallas_call(
        _attn_kernel,
        grid=(H2,),
        in_specs=[
            pl.BlockSpec((S, HB * D), lambda h: (0, h)),
            pl.BlockSpec((HB, C, D), lambda h: (h, 0, 0)),
            pl.BlockSpec((S, HB * D), lambda h: (0, h)),
        ],
        out_specs=[
            pl.BlockSpec((HB, S, C), lambda h: (h, 0, 0)),
            pl.BlockSpec((HB, C, D), lambda h: (h, 0, 0)),
        ],
        out_shape=[
            jax.ShapeDtypeStruct((H, S, C), f32),
            jax.ShapeDtypeStruct((H, C, D), f32),
        ],
    )(q_, k_, v_)

    o = pl.pallas_call(
        _out_kernel,
        out_shape=jax.ShapeDtypeStruct((C, IN), f32),
    )(values, W_p, b_p2, g_o2, bb_o2)

    return (o, c, attn, topk_idx)
